# baseline (device time: 30739 ns/iter reference)
import jax
import jax.numpy as jnp
from jax import lax
from jax.experimental import pallas as pl
from jax.experimental.pallas import tpu as pltpu

N_DEV = 16
N_STEPS = 4
EXPERTS_PER_DEV = 2


def kernel(x, router_W, route_idx, expert_W, shared_W):
    n_tok, d_model = x.shape
    n_experts = router_W.shape[1]
    d_ff = expert_W.shape[2]

    def body(x_ref, router_W_ref, route_idx_ref, expert_W_ref, shared_W_ref,
             out_ref, acc_ref, comm_ref, send_sems, recv_sems):
        my = lax.axis_index("i")

        barrier_sem = pltpu.get_barrier_semaphore()
        for k in range(N_STEPS):
            partner = my ^ (1 << k)
            pl.semaphore_signal(
                barrier_sem, inc=1,
                device_id=(partner,), device_id_type=pl.DeviceIdType.MESH,
            )
        pl.semaphore_wait(barrier_sem, N_STEPS)

        xv = x_ref[...]
        xb = xv.astype(jnp.bfloat16)
        scores = jnp.dot(xb, router_W_ref[...].astype(jnp.bfloat16),
                         preferred_element_type=jnp.float32)
        s_max = jnp.max(scores, axis=-1, keepdims=True)
        p = jnp.exp(scores - s_max)
        probs = p / jnp.sum(p, axis=-1, keepdims=True)

        idx = route_idx_ref[...]
        eids = lax.broadcasted_iota(jnp.int32, (n_tok, n_experts), 1)
        p_tok = jnp.sum(jnp.where(idx == eids, probs, 0.0),
                        axis=-1, keepdims=True)

        partial = jnp.zeros((n_tok, d_ff), jnp.float32)
        for j in range(EXPERTS_PER_DEV):
            e_glob = my * EXPERTS_PER_DEV + j
            y = jnp.dot(xb, expert_W_ref[j].astype(jnp.bfloat16),
                        preferred_element_type=jnp.float32)
            coef = jnp.where(idx == e_glob, p_tok, 0.0)
            partial = partial + coef * y
        acc_ref[...] = partial

        for k in range(N_STEPS):
            partner = my ^ (1 << k)
            rdma = pltpu.make_async_remote_copy(
                src_ref=acc_ref,
                dst_ref=comm_ref.at[k],
                send_sem=send_sems.at[k],
                recv_sem=recv_sems.at[k],
                device_id=(partner,),
                device_id_type=pl.DeviceIdType.MESH,
            )
            rdma.start()
            rdma.wait()
            acc_ref[...] = acc_ref[...] + comm_ref[k]

        shared = jnp.dot(xb, shared_W_ref[...].astype(jnp.bfloat16),
                         preferred_element_type=jnp.float32)
        out_ref[...] = acc_ref[...] + shared

    return pl.pallas_call(
        body,
        out_shape=jax.ShapeDtypeStruct((n_tok, d_ff), jnp.float32),
        in_specs=[pl.BlockSpec(memory_space=pltpu.VMEM)] * 5,
        out_specs=pl.BlockSpec(memory_space=pltpu.VMEM),
        scratch_shapes=[
            pltpu.VMEM((n_tok, d_ff), jnp.float32),
            pltpu.VMEM((N_STEPS, n_tok, d_ff), jnp.float32),
            pltpu.SemaphoreType.DMA((N_STEPS,)),
            pltpu.SemaphoreType.DMA((N_STEPS,)),
        ],
        compiler_params=pltpu.CompilerParams(collective_id=0),
    )(x, router_W, route_idx, expert_W, shared_W)


# device time: 23473 ns/iter; 1.3095x vs baseline; 1.3095x over previous
import jax
import jax.numpy as jnp
from jax import lax
from jax.experimental import pallas as pl
from jax.experimental.pallas import tpu as pltpu

N_DEV = 16
N_STEPS = 4
EXPERTS_PER_DEV = 2


def kernel(x, router_W, route_idx, expert_W, shared_W):
    n_tok, d_model = x.shape
    n_experts = router_W.shape[1]
    d_ff = expert_W.shape[2]

    def body(x_ref, router_W_ref, route_idx_ref, expert_W_ref, shared_W_ref,
             out_ref, acc_ref, send_ref, comm_ref, send_sems, recv_sems):
        my = lax.axis_index("i")

        barrier_sem = pltpu.get_barrier_semaphore()
        for k in range(N_STEPS):
            partner = my ^ (1 << k)
            pl.semaphore_signal(
                barrier_sem, inc=1,
                device_id=(partner,), device_id_type=pl.DeviceIdType.MESH,
            )
        pl.semaphore_wait(barrier_sem, N_STEPS)

        xv = x_ref[...]
        xb = xv.astype(jnp.bfloat16)
        scores = jnp.dot(xb, router_W_ref[...].astype(jnp.bfloat16),
                         preferred_element_type=jnp.float32)
        s_max = jnp.max(scores, axis=-1, keepdims=True)
        p = jnp.exp(scores - s_max)
        probs = p / jnp.sum(p, axis=-1, keepdims=True)

        idx = route_idx_ref[...]
        eids = lax.broadcasted_iota(jnp.int32, (n_tok, n_experts), 1)
        p_tok = jnp.sum(jnp.where(idx == eids, probs, 0.0),
                        axis=-1, keepdims=True)

        partial = jnp.zeros((n_tok, d_ff), jnp.float32)
        for j in range(EXPERTS_PER_DEV):
            e_glob = my * EXPERTS_PER_DEV + j
            y = jnp.dot(xb, expert_W_ref[j].astype(jnp.bfloat16),
                        preferred_element_type=jnp.float32)
            coef = jnp.where(idx == e_glob, p_tok, 0.0)
            partial = partial + coef * y
        acc_ref[...] = partial

        for k in range(N_STEPS):
            partner = my ^ (1 << k)
            send_ref[k] = acc_ref[...].astype(jnp.bfloat16)
            rdma = pltpu.make_async_remote_copy(
                src_ref=send_ref.at[k],
                dst_ref=comm_ref.at[k],
                send_sem=send_sems.at[k],
                recv_sem=recv_sems.at[k],
                device_id=(partner,),
                device_id_type=pl.DeviceIdType.MESH,
            )
            rdma.start()
            if k == 0:
                shared = jnp.dot(xb, shared_W_ref[...].astype(jnp.bfloat16),
                                 preferred_element_type=jnp.float32)
                out_ref[...] = shared
            rdma.wait()
            acc_ref[...] = acc_ref[...] + comm_ref[k].astype(jnp.float32)

        out_ref[...] = out_ref[...] + acc_ref[...]

    return pl.pallas_call(
        body,
        out_shape=jax.ShapeDtypeStruct((n_tok, d_ff), jnp.float32),
        in_specs=[pl.BlockSpec(memory_space=pltpu.VMEM)] * 5,
        out_specs=pl.BlockSpec(memory_space=pltpu.VMEM),
        scratch_shapes=[
            pltpu.VMEM((n_tok, d_ff), jnp.float32),
            pltpu.VMEM((N_STEPS, n_tok, d_ff), jnp.bfloat16),
            pltpu.VMEM((N_STEPS, n_tok, d_ff), jnp.bfloat16),
            pltpu.SemaphoreType.DMA((N_STEPS,)),
            pltpu.SemaphoreType.DMA((N_STEPS,)),
        ],
        compiler_params=pltpu.CompilerParams(collective_id=0),
    )(x, router_W, route_idx, expert_W, shared_W)


# device time: 7430 ns/iter; 4.1371x vs baseline; 3.1592x over previous
import jax
import jax.numpy as jnp
from jax import lax
from jax.experimental import pallas as pl
from jax.experimental.pallas import tpu as pltpu

N_DEV = 16
N_STEPS = 4
EXPERTS_PER_DEV = 2


def kernel(x, router_W, route_idx, expert_W, shared_W):
    n_tok, d_model = x.shape
    n_experts = router_W.shape[1]
    d_ff = expert_W.shape[2]

    def body(x_ref, router_W_ref, route_idx_ref, expert_W_ref, shared_W_ref,
             out_ref, acc_ref, send_ref, comm_ref, send_sems, recv_sems):
        my = lax.axis_index("i")

        barrier_sem = pltpu.get_barrier_semaphore()
        for k in range(N_STEPS):
            partner = my ^ (1 << k)
            pl.semaphore_signal(
                barrier_sem, inc=1,
                device_id=(partner,), device_id_type=pl.DeviceIdType.MESH,
            )

        xv = x_ref[...]
        xb = xv.astype(jnp.bfloat16)
        scores = jnp.dot(xb, router_W_ref[...].astype(jnp.bfloat16),
                         preferred_element_type=jnp.float32)
        s_max = jnp.max(scores, axis=-1, keepdims=True)
        p = jnp.exp(scores - s_max)
        probs = p / jnp.sum(p, axis=-1, keepdims=True)

        idx = route_idx_ref[...]
        eids = lax.broadcasted_iota(jnp.int32, (n_tok, n_experts), 1)
        p_tok = jnp.sum(jnp.where(idx == eids, probs, 0.0),
                        axis=-1, keepdims=True)

        partial = jnp.zeros((n_tok, d_ff), jnp.float32)
        for j in range(EXPERTS_PER_DEV):
            e_glob = my * EXPERTS_PER_DEV + j
            y = jnp.dot(xb, expert_W_ref[j].astype(jnp.bfloat16),
                        preferred_element_type=jnp.float32)
            coef = jnp.where(idx == e_glob, p_tok, 0.0)
            partial = partial + coef * y
        acc_ref[...] = partial
        send_ref[0] = partial.astype(jnp.bfloat16)

        pl.semaphore_wait(barrier_sem, N_STEPS)

        rdmas = []
        for k in range(N_STEPS):
            partner = my ^ (1 << k)
            rdma = pltpu.make_async_remote_copy(
                src_ref=send_ref.at[k],
                dst_ref=comm_ref.at[k],
                send_sem=send_sems.at[k],
                recv_sem=recv_sems.at[k],
                device_id=(partner,),
                device_id_type=pl.DeviceIdType.MESH,
            )
            rdma.start()
            rdmas.append(rdma)
            if k == 0:
                shared = jnp.dot(xb, shared_W_ref[...].astype(jnp.bfloat16),
                                 preferred_element_type=jnp.float32)
                out_ref[...] = shared
            rdma.wait_recv()
            new_acc = acc_ref[...] + comm_ref[k].astype(jnp.float32)
            acc_ref[...] = new_acc
            if k + 1 < N_STEPS:
                send_ref[k + 1] = new_acc.astype(jnp.bfloat16)

        out_ref[...] = out_ref[...] + acc_ref[...]
        for rdma in rdmas:
            rdma.wait_send()

    return pl.pallas_call(
        body,
        out_shape=jax.ShapeDtypeStruct((n_tok, d_ff), jnp.float32),
        in_specs=[pl.BlockSpec(memory_space=pltpu.VMEM)] * 5,
        out_specs=pl.BlockSpec(memory_space=pltpu.VMEM),
        scratch_shapes=[
            pltpu.VMEM((n_tok, d_ff), jnp.float32),
            pltpu.VMEM((N_STEPS, n_tok, d_ff), jnp.bfloat16),
            pltpu.VMEM((N_STEPS, n_tok, d_ff), jnp.bfloat16),
            pltpu.SemaphoreType.DMA((N_STEPS,)),
            pltpu.SemaphoreType.DMA((N_STEPS,)),
        ],
        compiler_params=pltpu.CompilerParams(collective_id=0),
    )(x, router_W, route_idx, expert_W, shared_W)
